# baseline (speedup 1.0000x reference)
import functools

import jax
import jax.numpy as jnp
from jax import lax
from jax.experimental import pallas as pl
from jax.experimental.pallas import tpu as pltpu
from jax.experimental.pallas import tpu_sc as plsc

DIM = 64
NC = 2
NS = 16
NW = NC * NS
N_BUF = 4


@functools.cache
def _make_kernel(NI, NJ):
    assert NI % (NW * N_BUF) == 0
    rows_per_w = NI // NW
    n_outer = rows_per_w // N_BUF
    mesh = plsc.VectorSubcoreMesh(core_axis_name="c", subcore_axis_name="s")

    @functools.partial(
        pl.kernel,
        mesh=mesh,
        out_type=jax.ShapeDtypeStruct((NI, NJ, DIM), jnp.float32),
        scratch_types=[
            pltpu.VMEM((rows_per_w, NJ), jnp.int32),
            pltpu.VMEM((N_BUF, NJ, DIM), jnp.float32),
        ] + [pltpu.SemaphoreType.DMA] * (2 * N_BUF),
        compiler_params=pltpu.CompilerParams(use_tc_tiling_on_sc=False),
    )
    def gather_kernel(idx_hbm, table_hbm, out_hbm, idx_v, rows_v, *sems):
        gsem, wsem = sems[:N_BUF], sems[N_BUF:]
        wid = lax.axis_index("s") * NC + lax.axis_index("c")
        i_base = wid * rows_per_w
        pltpu.sync_copy(idx_hbm.at[pl.ds(i_base, rows_per_w)], idx_v)

        for b in range(N_BUF):
            pltpu.async_copy(table_hbm.at[idx_v.at[b]], rows_v.at[b], gsem[b])

        def body(t, carry):
            r0 = t * N_BUF
            for b in range(N_BUF):
                pltpu.make_async_copy(
                    table_hbm.at[idx_v.at[r0 + b]], rows_v.at[b], gsem[b]
                ).wait()
                pltpu.async_copy(rows_v.at[b], out_hbm.at[i_base + r0 + b], wsem[b])

            @pl.when(t < n_outer - 1)
            def _refill():
                for b in range(N_BUF):
                    pltpu.make_async_copy(
                        rows_v.at[b], out_hbm.at[i_base + r0 + b], wsem[b]
                    ).wait()
                    pltpu.async_copy(
                        table_hbm.at[idx_v.at[r0 + N_BUF + b]], rows_v.at[b], gsem[b]
                    )

            return carry

        lax.fori_loop(0, n_outer, body, 0)

        for b in range(N_BUF):
            r = rows_per_w - N_BUF + b
            pltpu.make_async_copy(
                rows_v.at[b], out_hbm.at[i_base + r], wsem[b]
            ).wait()

    return gather_kernel


def kernel(edges, weight):
    NI, NJ = edges.shape
    return _make_kernel(NI, NJ)(edges.astype(jnp.int32), weight)


# speedup vs baseline: 1.0318x; 1.0318x over previous
import functools

import jax
import jax.numpy as jnp
from jax import lax
from jax.experimental import pallas as pl
from jax.experimental.pallas import tpu as pltpu
from jax.experimental.pallas import tpu_sc as plsc

DIM = 64
CHUNK = 128
NC = 2
NS = 16
NW = NC * NS
N_BUF = 4


@functools.cache
def _make_kernel(B):
    assert B % (NW * CHUNK * N_BUF) == 0
    chunks_per_w = B // (NW * CHUNK)
    n_outer = chunks_per_w // N_BUF
    mesh = plsc.VectorSubcoreMesh(core_axis_name="c", subcore_axis_name="s")

    @functools.partial(
        pl.kernel,
        mesh=mesh,
        out_type=jax.ShapeDtypeStruct((B, DIM), jnp.float32),
        scratch_types=[
            pltpu.VMEM((chunks_per_w, CHUNK), jnp.int32),
            pltpu.VMEM((N_BUF, CHUNK, DIM), jnp.float32),
        ] + [pltpu.SemaphoreType.DMA] * (2 * N_BUF),
        compiler_params=pltpu.CompilerParams(use_tc_tiling_on_sc=False),
    )
    def gather_kernel(idx_hbm, table_hbm, out_hbm, idx_v, rows_v, *sems):
        gsem, wsem = sems[:N_BUF], sems[N_BUF:]
        wid = lax.axis_index("s") * NC + lax.axis_index("c")
        chunk_base = wid * chunks_per_w
        pltpu.sync_copy(idx_hbm.at[pl.ds(chunk_base, chunks_per_w)], idx_v)
        row_base = chunk_base * CHUNK

        def out_slice(j):
            return out_hbm.at[pl.ds(row_base + j * CHUNK, CHUNK)]

        for b in range(N_BUF):
            pltpu.async_copy(table_hbm.at[idx_v.at[b]], rows_v.at[b], gsem[b])

        def body(i, carry):
            j0 = i * N_BUF
            for b in range(N_BUF):
                pltpu.make_async_copy(
                    table_hbm.at[idx_v.at[j0 + b]], rows_v.at[b], gsem[b]
                ).wait()
                pltpu.async_copy(rows_v.at[b], out_slice(j0 + b), wsem[b])

            @pl.when(i < n_outer - 1)
            def _refill():
                for b in range(N_BUF):
                    pltpu.make_async_copy(
                        rows_v.at[b], out_slice(j0 + b), wsem[b]
                    ).wait()
                    pltpu.async_copy(
                        table_hbm.at[idx_v.at[j0 + N_BUF + b]], rows_v.at[b], gsem[b]
                    )

            return carry

        lax.fori_loop(0, n_outer, body, 0)

        for b in range(N_BUF):
            j = chunks_per_w - N_BUF + b
            pltpu.make_async_copy(rows_v.at[b], out_slice(j), wsem[b]).wait()

    return gather_kernel


def kernel(edges, weight):
    nr, nc = edges.shape
    B = nr * nc
    idx = edges.reshape(B // CHUNK, CHUNK).astype(jnp.int32)
    out = _make_kernel(B)(idx, weight)
    return out.reshape(nr, nc, DIM)


# speedup vs baseline: 1.0340x; 1.0021x over previous
import functools

import jax
import jax.numpy as jnp
from jax import lax
from jax.experimental import pallas as pl
from jax.experimental.pallas import tpu as pltpu
from jax.experimental.pallas import tpu_sc as plsc

DIM = 64
CHUNK = 128
NC = 2
NS = 16
NW = NC * NS
N_BUF = 8


@functools.cache
def _make_kernel(B):
    assert B % (NW * CHUNK * N_BUF) == 0
    chunks_per_w = B // (NW * CHUNK)
    n_outer = chunks_per_w // N_BUF
    mesh = plsc.VectorSubcoreMesh(core_axis_name="c", subcore_axis_name="s")

    @functools.partial(
        pl.kernel,
        mesh=mesh,
        out_type=jax.ShapeDtypeStruct((B, DIM), jnp.float32),
        scratch_types=[
            pltpu.VMEM((chunks_per_w, CHUNK), jnp.int32),
            pltpu.VMEM((N_BUF, CHUNK, DIM), jnp.float32),
        ] + [pltpu.SemaphoreType.DMA] * (2 * N_BUF),
        compiler_params=pltpu.CompilerParams(use_tc_tiling_on_sc=False),
    )
    def gather_kernel(idx_hbm, table_hbm, out_hbm, idx_v, rows_v, *sems):
        gsem, wsem = sems[:N_BUF], sems[N_BUF:]
        wid = lax.axis_index("s") * NC + lax.axis_index("c")
        chunk_base = wid * chunks_per_w
        pltpu.sync_copy(idx_hbm.at[pl.ds(chunk_base, chunks_per_w)], idx_v)
        row_base = chunk_base * CHUNK

        def out_slice(j):
            return out_hbm.at[pl.ds(row_base + j * CHUNK, CHUNK)]

        for b in range(N_BUF):
            pltpu.async_copy(table_hbm.at[idx_v.at[b]], rows_v.at[b], gsem[b])

        def body(i, carry):
            j0 = i * N_BUF
            for b in range(N_BUF):
                pltpu.make_async_copy(
                    table_hbm.at[idx_v.at[j0 + b]], rows_v.at[b], gsem[b]
                ).wait()
                pltpu.async_copy(rows_v.at[b], out_slice(j0 + b), wsem[b])

            @pl.when(i < n_outer - 1)
            def _refill():
                for b in range(N_BUF):
                    pltpu.make_async_copy(
                        rows_v.at[b], out_slice(j0 + b), wsem[b]
                    ).wait()
                    pltpu.async_copy(
                        table_hbm.at[idx_v.at[j0 + N_BUF + b]], rows_v.at[b], gsem[b]
                    )

            return carry

        lax.fori_loop(0, n_outer, body, 0)

        for b in range(N_BUF):
            j = chunks_per_w - N_BUF + b
            pltpu.make_async_copy(rows_v.at[b], out_slice(j), wsem[b]).wait()

    return gather_kernel


def kernel(edges, weight):
    nr, nc = edges.shape
    B = nr * nc
    idx = edges.reshape(B // CHUNK, CHUNK).astype(jnp.int32)
    out = _make_kernel(B)(idx, weight)
    return out.reshape(nr, nc, DIM)
